# baseline (speedup 1.0000x reference)
import functools

import jax
import jax.numpy as jnp
from jax import lax
from jax.experimental import pallas as pl
from jax.experimental.pallas import tpu as pltpu
from jax.experimental.pallas import tpu_sc as plsc

N = 10000
E = 320000
D = 128

NC = 2
NS = 16
NW = NC * NS
CH = 128
NCHUNK = 80
EPAD = NW * NCHUNK * CH
NBUF = 1
NACC = N + 8
RMAIN = 624
RTAIL_BASE = NS * RMAIN

_SC_MESH = plsc.VectorSubcoreMesh(
    core_axis_name="c", subcore_axis_name="s", num_cores=NC, num_subcores=NS)


def _tilewise_copy(src, dsti, s):
  pltpu.sync_copy(src.at[pl.ds(s * RMAIN, RMAIN)],
                  dsti.at[pl.ds(s * RMAIN, RMAIN)])

  @pl.when(s < 2)
  def _():
    pltpu.sync_copy(src.at[pl.ds(RTAIL_BASE + s * 8, 8)],
                    dsti.at[pl.ds(RTAIL_BASE + s * 8, 8)])


def _deg_body(dst3_hbm, ones_hbm, out0_hbm, out1_hbm,
              didx2, ones_v, buf_v, acc_sh):
  c = lax.axis_index("c")
  s = lax.axis_index("s")
  wid = c * NS + s

  def zfill(i, carry):
    buf_v[pl.ds(i * 16, 16)] = jnp.zeros((16,), jnp.float32)
    return carry

  lax.fori_loop(0, RMAIN // 16, zfill, 0)
  pltpu.sync_copy(buf_v, acc_sh.at[pl.ds(s * RMAIN, RMAIN)])

  @pl.when(s < 2)
  def _():
    pltpu.sync_copy(buf_v.at[pl.ds(0, 8)],
                    acc_sh.at[pl.ds(RTAIL_BASE + s * 8, 8)])

  pltpu.sync_copy(dst3_hbm.at[wid], didx2)
  pltpu.sync_copy(ones_hbm, ones_v)
  plsc.subcore_barrier()

  def chunk(i, carry):
    pltpu.sync_copy(ones_v, acc_sh.at[didx2.at[i]], add=True)
    return carry

  lax.fori_loop(0, NCHUNK, chunk, 0)
  plsc.subcore_barrier()

  @pl.when(c == 0)
  def _():
    pltpu.sync_copy(acc_sh.at[pl.ds(s * RMAIN, RMAIN)], buf_v)
    pltpu.sync_copy(buf_v, out0_hbm.at[pl.ds(s * RMAIN, RMAIN)])

    @pl.when(s < 2)
    def _():
      pltpu.sync_copy(acc_sh.at[pl.ds(RTAIL_BASE + s * 8, 8)],
                      buf_v.at[pl.ds(0, 8)])
      pltpu.sync_copy(buf_v.at[pl.ds(0, 8)],
                      out0_hbm.at[pl.ds(RTAIL_BASE + s * 8, 8)])

  @pl.when(c == 1)
  def _():
    pltpu.sync_copy(acc_sh.at[pl.ds(s * RMAIN, RMAIN)], buf_v)
    pltpu.sync_copy(buf_v, out1_hbm.at[pl.ds(s * RMAIN, RMAIN)])

    @pl.when(s < 2)
    def _():
      pltpu.sync_copy(acc_sh.at[pl.ds(RTAIL_BASE + s * 8, 8)],
                      buf_v.at[pl.ds(0, 8)])
      pltpu.sync_copy(buf_v.at[pl.ds(0, 8)],
                      out1_hbm.at[pl.ds(RTAIL_BASE + s * 8, 8)])


_deg_call = pl.kernel(
    _deg_body,
    out_type=(jax.ShapeDtypeStruct((N,), jnp.float32),
              jax.ShapeDtypeStruct((N,), jnp.float32)),
    mesh=_SC_MESH,
    scratch_types=[
        pltpu.VMEM((NCHUNK, CH), jnp.int32),
        pltpu.VMEM((CH,), jnp.float32),
        pltpu.VMEM((RMAIN,), jnp.float32),
        pltpu.VMEM_SHARED((NACC,), jnp.float32),
    ],
)


def _agg_body(hp_hbm, src3_hbm, dst3_hbm, zeros_hbm, out_hbm,
              sidx2, didx2, rows_bufs, sems, acc_sh):
  c = lax.axis_index("c")
  s = lax.axis_index("s")
  wid = c * NS + s
  pltpu.sync_copy(src3_hbm.at[wid], sidx2)
  pltpu.sync_copy(dst3_hbm.at[wid], didx2)
  _tilewise_copy(zeros_hbm, acc_sh, s)
  plsc.subcore_barrier()

  for b in range(NBUF):
    pltpu.async_copy(hp_hbm.at[sidx2.at[b]], rows_bufs[b], sems[b])

  def outer(g, carry):
    j0 = g * NBUF
    for b in range(NBUF):
      j = j0 + b
      rb = rows_bufs[b]
      pltpu.make_async_copy(hp_hbm.at[sidx2.at[j]], rb, sems[b]).wait()
      pltpu.sync_copy(rb, acc_sh.at[didx2.at[j]], add=True)

      @pl.when(j + NBUF < NCHUNK)
      def _():
        pltpu.async_copy(hp_hbm.at[sidx2.at[j + NBUF]], rb, sems[b])
    return carry

  lax.fori_loop(0, NCHUNK // NBUF, outer, 0)
  plsc.subcore_barrier()
  _tilewise_copy(acc_sh, out_hbm.at[c], s)


_agg_call = pl.kernel(
    _agg_body,
    out_type=jax.ShapeDtypeStruct((NC, N, D), jnp.float32),
    mesh=_SC_MESH,
    scratch_types=[
        pltpu.VMEM((NCHUNK, CH), jnp.int32),
        pltpu.VMEM((NCHUNK, CH), jnp.int32),
        tuple(pltpu.VMEM((CH, D), jnp.float32) for _ in range(NBUF)),
        tuple(pltpu.SemaphoreType.DMA for _ in range(NBUF)),
        pltpu.VMEM_SHARED((NACC, D), jnp.float32),
    ],
)


_MM = functools.partial(jnp.dot, precision=lax.Precision.HIGHEST,
                        preferred_element_type=jnp.float32)


def _dinv(deg_ref):
  return lax.rsqrt(1.0 + deg_ref[...])


def _first_body(x_ref, w_ref, deg_ref, o_ref):
  o_ref[...] = _MM(x_ref[...], w_ref[...]) * _dinv(deg_ref)


def _mid_body(a_ref, hp_ref, deg_ref, b_ref, w_ref, o_ref):
  dinv = _dinv(deg_ref)
  pre = (a_ref[0] + a_ref[1] + hp_ref[...]) * dinv + b_ref[...]
  act = jnp.maximum(pre, 0.0)
  o_ref[...] = _MM(act, w_ref[...]) * dinv


def _last_body(a_ref, hp_ref, deg_ref, b_ref, o_ref):
  pre = (a_ref[0] + a_ref[1] + hp_ref[...]) * _dinv(deg_ref) + b_ref[...]
  m = jnp.max(pre, axis=1, keepdims=True)
  e = jnp.exp(pre - m)
  o_ref[...] = e / jnp.sum(e, axis=1, keepdims=True)


_f32 = jnp.float32
_first_call = pl.pallas_call(
    _first_body, out_shape=jax.ShapeDtypeStruct((N, D), _f32))
_mid_call = pl.pallas_call(
    _mid_body, out_shape=jax.ShapeDtypeStruct((N, D), _f32))
_last_call = pl.pallas_call(
    _last_body, out_shape=jax.ShapeDtypeStruct((N, D), _f32))


def kernel(x, edge_idx, W1, b1, W2, b2, W3, b3, W4, b4):
  src = edge_idx[0]
  dst = edge_idx[1]
  zerosD = jnp.zeros((N, D), jnp.float32)
  ones1 = jnp.ones((CH,), jnp.float32)

  npad = EPAD - E
  src3 = jnp.concatenate([src, jnp.zeros((npad,), jnp.int32)]).reshape(
      NW, NCHUNK, CH)
  dst3 = jnp.concatenate([dst, jnp.full((npad,), N, jnp.int32)]).reshape(
      NW, NCHUNK, CH)

  deg0, deg1 = _deg_call(dst3, ones1)
  deg = (deg0 + deg1).reshape(N, 1)

  h1 = _first_call(x, W1, deg)
  a1 = _agg_call(h1, src3, dst3, zerosD)
  h2 = _mid_call(a1, h1, deg, b1.reshape(1, D), W2)
  a2 = _agg_call(h2, src3, dst3, zerosD)
  h3 = _mid_call(a2, h2, deg, b2.reshape(1, D), W3)
  a3 = _agg_call(h3, src3, dst3, zerosD)
  h4 = _mid_call(a3, h3, deg, b3.reshape(1, D), W4)
  a4 = _agg_call(h4, src3, dst3, zerosD)
  return _last_call(a4, h4, deg, b4.reshape(1, D))


# speedup vs baseline: 1.2068x; 1.2068x over previous
import functools

import jax
import jax.numpy as jnp
from jax import lax
from jax.experimental import pallas as pl
from jax.experimental.pallas import tpu as pltpu
from jax.experimental.pallas import tpu_sc as plsc

N = 10000
E = 320000
D = 128

NC = 2
NS = 16
NW = NC * NS
CH = 128
NCHUNK = 80
EPAD = NW * NCHUNK * CH
NBUF = 1
NACC = N + 8
RMAIN = 624
RTAIL_BASE = NS * RMAIN

_SC_MESH = plsc.VectorSubcoreMesh(
    core_axis_name="c", subcore_axis_name="s", num_cores=NC, num_subcores=NS)


def _tilewise_copy(src, dsti, s):
  pltpu.sync_copy(src.at[pl.ds(s * RMAIN, RMAIN)],
                  dsti.at[pl.ds(s * RMAIN, RMAIN)])

  @pl.when(s < 2)
  def _():
    pltpu.sync_copy(src.at[pl.ds(RTAIL_BASE + s * 8, 8)],
                    dsti.at[pl.ds(RTAIL_BASE + s * 8, 8)])


def _deg_body(dst3_hbm, ones_hbm, out0_hbm, out1_hbm,
              didx2, ones_v, buf_v, acc_sh):
  c = lax.axis_index("c")
  s = lax.axis_index("s")
  wid = c * NS + s

  def zfill(i, carry):
    buf_v[pl.ds(i * 16, 16)] = jnp.zeros((16,), jnp.float32)
    return carry

  lax.fori_loop(0, RMAIN // 16, zfill, 0)
  pltpu.sync_copy(buf_v, acc_sh.at[pl.ds(s * RMAIN, RMAIN)])

  @pl.when(s < 2)
  def _():
    pltpu.sync_copy(buf_v.at[pl.ds(0, 8)],
                    acc_sh.at[pl.ds(RTAIL_BASE + s * 8, 8)])

  pltpu.sync_copy(dst3_hbm.at[wid], didx2)
  pltpu.sync_copy(ones_hbm, ones_v)
  plsc.subcore_barrier()

  def chunk(i, carry):
    pltpu.sync_copy(ones_v, acc_sh.at[didx2.at[i]], add=True)
    return carry

  lax.fori_loop(0, NCHUNK, chunk, 0)
  plsc.subcore_barrier()

  @pl.when(c == 0)
  def _():
    pltpu.sync_copy(acc_sh.at[pl.ds(s * RMAIN, RMAIN)], buf_v)
    pltpu.sync_copy(buf_v, out0_hbm.at[pl.ds(s * RMAIN, RMAIN)])

    @pl.when(s < 2)
    def _():
      pltpu.sync_copy(acc_sh.at[pl.ds(RTAIL_BASE + s * 8, 8)],
                      buf_v.at[pl.ds(0, 8)])
      pltpu.sync_copy(buf_v.at[pl.ds(0, 8)],
                      out0_hbm.at[pl.ds(RTAIL_BASE + s * 8, 8)])

  @pl.when(c == 1)
  def _():
    pltpu.sync_copy(acc_sh.at[pl.ds(s * RMAIN, RMAIN)], buf_v)
    pltpu.sync_copy(buf_v, out1_hbm.at[pl.ds(s * RMAIN, RMAIN)])

    @pl.when(s < 2)
    def _():
      pltpu.sync_copy(acc_sh.at[pl.ds(RTAIL_BASE + s * 8, 8)],
                      buf_v.at[pl.ds(0, 8)])
      pltpu.sync_copy(buf_v.at[pl.ds(0, 8)],
                      out1_hbm.at[pl.ds(RTAIL_BASE + s * 8, 8)])


_deg_call = pl.kernel(
    _deg_body,
    out_type=(jax.ShapeDtypeStruct((N,), jnp.float32),
              jax.ShapeDtypeStruct((N,), jnp.float32)),
    mesh=_SC_MESH,
    scratch_types=[
        pltpu.VMEM((NCHUNK, CH), jnp.int32),
        pltpu.VMEM((CH,), jnp.float32),
        pltpu.VMEM((RMAIN,), jnp.float32),
        pltpu.VMEM_SHARED((NACC,), jnp.float32),
    ],
)


def _agg_body(hp_hbm, src3_hbm, dst3_hbm, zeros_hbm, out_hbm,
              sidx2, didx2, rows_bufs, sems, acc_sh):
  c = lax.axis_index("c")
  s = lax.axis_index("s")
  wid = c * NS + s
  pltpu.sync_copy(src3_hbm.at[wid], sidx2)
  pltpu.sync_copy(dst3_hbm.at[wid], didx2)
  _tilewise_copy(zeros_hbm, acc_sh, s)
  plsc.subcore_barrier()

  for b in range(NBUF):
    pltpu.async_copy(hp_hbm.at[sidx2.at[b]], rows_bufs[b], sems[b])

  def outer(g, carry):
    j0 = g * NBUF
    for b in range(NBUF):
      j = j0 + b
      rb = rows_bufs[b]
      pltpu.make_async_copy(hp_hbm.at[sidx2.at[j]], rb, sems[b]).wait()
      pltpu.sync_copy(rb, acc_sh.at[didx2.at[j]], add=True)

      @pl.when(j + NBUF < NCHUNK)
      def _():
        pltpu.async_copy(hp_hbm.at[sidx2.at[j + NBUF]], rb, sems[b])
    return carry

  lax.fori_loop(0, NCHUNK // NBUF, outer, 0)
  plsc.subcore_barrier()
  _tilewise_copy(acc_sh, out_hbm.at[c], s)


_agg_call = pl.kernel(
    _agg_body,
    out_type=jax.ShapeDtypeStruct((NC, N, D), jnp.float32),
    mesh=_SC_MESH,
    scratch_types=[
        pltpu.VMEM((NCHUNK, CH), jnp.int32),
        pltpu.VMEM((NCHUNK, CH), jnp.int32),
        tuple(pltpu.VMEM((CH, D), jnp.float32) for _ in range(NBUF)),
        tuple(pltpu.SemaphoreType.DMA for _ in range(NBUF)),
        pltpu.VMEM_SHARED((NACC, D), jnp.float32),
    ],
)


_MM = functools.partial(jnp.dot, precision=lax.Precision.HIGHEST,
                        preferred_element_type=jnp.float32)


def _dinv(deg_ref):
  return lax.rsqrt(1.0 + deg_ref[...])


def _first_body(x_ref, w_ref, deg_ref, o_ref):
  o_ref[...] = _MM(x_ref[...], w_ref[...]) * _dinv(deg_ref)


def _mid_body(a_ref, hp_ref, deg_ref, b_ref, w_ref, o_ref):
  dinv = _dinv(deg_ref)
  pre = (a_ref[0] + a_ref[1] + hp_ref[...]) * dinv + b_ref[...]
  act = jnp.maximum(pre, 0.0)
  o_ref[...] = _MM(act, w_ref[...]) * dinv


def _last_body(a_ref, hp_ref, deg_ref, b_ref, o_ref):
  pre = (a_ref[0] + a_ref[1] + hp_ref[...]) * _dinv(deg_ref) + b_ref[...]
  m = jnp.max(pre, axis=1, keepdims=True)
  e = jnp.exp(pre - m)
  o_ref[...] = e / jnp.sum(e, axis=1, keepdims=True)


_f32 = jnp.float32
_first_call = pl.pallas_call(
    _first_body, out_shape=jax.ShapeDtypeStruct((N, D), _f32))
_mid_call = pl.pallas_call(
    _mid_body, out_shape=jax.ShapeDtypeStruct((N, D), _f32))
_last_call = pl.pallas_call(
    _last_body, out_shape=jax.ShapeDtypeStruct((N, D), _f32))


def kernel(x, edge_idx, W1, b1, W2, b2, W3, b3, W4, b4):
  src = edge_idx[0]
  dst = edge_idx[1]
  zerosD = jnp.zeros((N, D), jnp.float32)
  ones1 = jnp.ones((CH,), jnp.float32)

  epw = E // NW
  ndum = NCHUNK * CH - epw
  dum_dst = N + (jnp.arange(ndum, dtype=jnp.int32) % 8)
  src3 = jnp.concatenate(
      [src.reshape(NW, epw), jnp.zeros((NW, ndum), jnp.int32)],
      axis=1).reshape(NW, NCHUNK, CH)
  dst3 = jnp.concatenate(
      [dst.reshape(NW, epw), jnp.broadcast_to(dum_dst, (NW, ndum))],
      axis=1).reshape(NW, NCHUNK, CH)

  deg0, deg1 = _deg_call(dst3, ones1)
  deg = (deg0 + deg1).reshape(N, 1)

  h1 = _first_call(x, W1, deg)
  a1 = _agg_call(h1, src3, dst3, zerosD)
  h2 = _mid_call(a1, h1, deg, b1.reshape(1, D), W2)
  a2 = _agg_call(h2, src3, dst3, zerosD)
  h3 = _mid_call(a2, h2, deg, b2.reshape(1, D), W3)
  a3 = _agg_call(h3, src3, dst3, zerosD)
  h4 = _mid_call(a3, h3, deg, b3.reshape(1, D), W4)
  a4 = _agg_call(h4, src3, dst3, zerosD)
  return _last_call(a4, h4, deg, b4.reshape(1, D))


# speedup vs baseline: 1.2074x; 1.0005x over previous
import functools

import jax
import jax.numpy as jnp
from jax import lax
from jax.experimental import pallas as pl
from jax.experimental.pallas import tpu as pltpu
from jax.experimental.pallas import tpu_sc as plsc

N = 10000
E = 320000
D = 128

NC = 2
NS = 16
NW = NC * NS
CH = 128
NCHUNK = 80
EPAD = NW * NCHUNK * CH
NBUF = 1
NACC = N + 256
RMAIN = 624
RTAIL_BASE = NS * RMAIN

_SC_MESH = plsc.VectorSubcoreMesh(
    core_axis_name="c", subcore_axis_name="s", num_cores=NC, num_subcores=NS)


def _tilewise_copy(src, dsti, s):
  pltpu.sync_copy(src.at[pl.ds(s * RMAIN, RMAIN)],
                  dsti.at[pl.ds(s * RMAIN, RMAIN)])

  @pl.when(s < 2)
  def _():
    pltpu.sync_copy(src.at[pl.ds(RTAIL_BASE + s * 8, 8)],
                    dsti.at[pl.ds(RTAIL_BASE + s * 8, 8)])


def _deg_body(dst3_hbm, ones_hbm, out0_hbm, out1_hbm,
              didx2, ones_v, buf_v, acc_sh):
  c = lax.axis_index("c")
  s = lax.axis_index("s")
  wid = c * NS + s

  def zfill(i, carry):
    buf_v[pl.ds(i * 16, 16)] = jnp.zeros((16,), jnp.float32)
    return carry

  lax.fori_loop(0, RMAIN // 16, zfill, 0)
  pltpu.sync_copy(buf_v, acc_sh.at[pl.ds(s * RMAIN, RMAIN)])

  @pl.when(s < 2)
  def _():
    pltpu.sync_copy(buf_v.at[pl.ds(0, 8)],
                    acc_sh.at[pl.ds(RTAIL_BASE + s * 8, 8)])

  pltpu.sync_copy(dst3_hbm.at[wid], didx2)
  pltpu.sync_copy(ones_hbm, ones_v)
  plsc.subcore_barrier()

  def chunk(i, carry):
    pltpu.sync_copy(ones_v, acc_sh.at[didx2.at[i]], add=True)
    return carry

  lax.fori_loop(0, NCHUNK, chunk, 0)
  plsc.subcore_barrier()

  @pl.when(c == 0)
  def _():
    pltpu.sync_copy(acc_sh.at[pl.ds(s * RMAIN, RMAIN)], buf_v)
    pltpu.sync_copy(buf_v, out0_hbm.at[pl.ds(s * RMAIN, RMAIN)])

    @pl.when(s < 2)
    def _():
      pltpu.sync_copy(acc_sh.at[pl.ds(RTAIL_BASE + s * 8, 8)],
                      buf_v.at[pl.ds(0, 8)])
      pltpu.sync_copy(buf_v.at[pl.ds(0, 8)],
                      out0_hbm.at[pl.ds(RTAIL_BASE + s * 8, 8)])

  @pl.when(c == 1)
  def _():
    pltpu.sync_copy(acc_sh.at[pl.ds(s * RMAIN, RMAIN)], buf_v)
    pltpu.sync_copy(buf_v, out1_hbm.at[pl.ds(s * RMAIN, RMAIN)])

    @pl.when(s < 2)
    def _():
      pltpu.sync_copy(acc_sh.at[pl.ds(RTAIL_BASE + s * 8, 8)],
                      buf_v.at[pl.ds(0, 8)])
      pltpu.sync_copy(buf_v.at[pl.ds(0, 8)],
                      out1_hbm.at[pl.ds(RTAIL_BASE + s * 8, 8)])


_deg_call = pl.kernel(
    _deg_body,
    out_type=(jax.ShapeDtypeStruct((N,), jnp.float32),
              jax.ShapeDtypeStruct((N,), jnp.float32)),
    mesh=_SC_MESH,
    scratch_types=[
        pltpu.VMEM((NCHUNK, CH), jnp.int32),
        pltpu.VMEM((CH,), jnp.float32),
        pltpu.VMEM((RMAIN,), jnp.float32),
        pltpu.VMEM_SHARED((NACC,), jnp.float32),
    ],
)


def _agg_body(hp_hbm, src3_hbm, dst3_hbm, zeros_hbm, out_hbm,
              sidx2, didx2, rows_bufs, sems, acc_sh):
  c = lax.axis_index("c")
  s = lax.axis_index("s")
  wid = c * NS + s
  pltpu.sync_copy(src3_hbm.at[wid], sidx2)
  pltpu.sync_copy(dst3_hbm.at[wid], didx2)
  _tilewise_copy(zeros_hbm, acc_sh, s)
  plsc.subcore_barrier()

  for b in range(NBUF):
    pltpu.async_copy(hp_hbm.at[sidx2.at[b]], rows_bufs[b], sems[b])

  def outer(g, carry):
    j0 = g * NBUF
    for b in range(NBUF):
      j = j0 + b
      rb = rows_bufs[b]
      pltpu.make_async_copy(hp_hbm.at[sidx2.at[j]], rb, sems[b]).wait()
      pltpu.sync_copy(rb, acc_sh.at[didx2.at[j]], add=True)

      @pl.when(j + NBUF < NCHUNK)
      def _():
        pltpu.async_copy(hp_hbm.at[sidx2.at[j + NBUF]], rb, sems[b])
    return carry

  lax.fori_loop(0, NCHUNK // NBUF, outer, 0)
  plsc.subcore_barrier()
  _tilewise_copy(acc_sh, out_hbm.at[c], s)


_agg_call = pl.kernel(
    _agg_body,
    out_type=jax.ShapeDtypeStruct((NC, N, D), jnp.float32),
    mesh=_SC_MESH,
    scratch_types=[
        pltpu.VMEM((NCHUNK, CH), jnp.int32),
        pltpu.VMEM((NCHUNK, CH), jnp.int32),
        tuple(pltpu.VMEM((CH, D), jnp.float32) for _ in range(NBUF)),
        tuple(pltpu.SemaphoreType.DMA for _ in range(NBUF)),
        pltpu.VMEM_SHARED((NACC, D), jnp.float32),
    ],
)


_MM = functools.partial(jnp.dot, precision=lax.Precision.HIGHEST,
                        preferred_element_type=jnp.float32)


def _dinv(deg_ref):
  return lax.rsqrt(1.0 + deg_ref[...])


def _first_body(x_ref, w_ref, deg_ref, o_ref):
  o_ref[...] = _MM(x_ref[...], w_ref[...]) * _dinv(deg_ref)


def _mid_body(a_ref, hp_ref, deg_ref, b_ref, w_ref, o_ref):
  dinv = _dinv(deg_ref)
  pre = (a_ref[0] + a_ref[1] + hp_ref[...]) * dinv + b_ref[...]
  act = jnp.maximum(pre, 0.0)
  o_ref[...] = _MM(act, w_ref[...]) * dinv


def _last_body(a_ref, hp_ref, deg_ref, b_ref, o_ref):
  pre = (a_ref[0] + a_ref[1] + hp_ref[...]) * _dinv(deg_ref) + b_ref[...]
  m = jnp.max(pre, axis=1, keepdims=True)
  e = jnp.exp(pre - m)
  o_ref[...] = e / jnp.sum(e, axis=1, keepdims=True)


_f32 = jnp.float32
_first_call = pl.pallas_call(
    _first_body, out_shape=jax.ShapeDtypeStruct((N, D), _f32))
_mid_call = pl.pallas_call(
    _mid_body, out_shape=jax.ShapeDtypeStruct((N, D), _f32))
_last_call = pl.pallas_call(
    _last_body, out_shape=jax.ShapeDtypeStruct((N, D), _f32))


def kernel(x, edge_idx, W1, b1, W2, b2, W3, b3, W4, b4):
  src = edge_idx[0]
  dst = edge_idx[1]
  zerosD = jnp.zeros((N, D), jnp.float32)
  ones1 = jnp.ones((CH,), jnp.float32)

  epw = E // NW
  ndum = NCHUNK * CH - epw
  widx = jnp.arange(NW, dtype=jnp.int32) % NS
  dum_dst = (N + widx[:, None] * 16
             + (jnp.arange(ndum, dtype=jnp.int32) % 16)[None, :])
  src3 = jnp.concatenate(
      [src.reshape(NW, epw), jnp.zeros((NW, ndum), jnp.int32)],
      axis=1).reshape(NW, NCHUNK, CH)
  dst3 = jnp.concatenate(
      [dst.reshape(NW, epw), dum_dst],
      axis=1).reshape(NW, NCHUNK, CH)

  deg0, deg1 = _deg_call(dst3, ones1)
  deg = (deg0 + deg1).reshape(N, 1)

  h1 = _first_call(x, W1, deg)
  a1 = _agg_call(h1, src3, dst3, zerosD)
  h2 = _mid_call(a1, h1, deg, b1.reshape(1, D), W2)
  a2 = _agg_call(h2, src3, dst3, zerosD)
  h3 = _mid_call(a2, h2, deg, b2.reshape(1, D), W3)
  a3 = _agg_call(h3, src3, dst3, zerosD)
  h4 = _mid_call(a3, h3, deg, b3.reshape(1, D), W4)
  a4 = _agg_call(h4, src3, dst3, zerosD)
  return _last_call(a4, h4, deg, b4.reshape(1, D))


# speedup vs baseline: 2.8351x; 2.3480x over previous
import functools

import jax
import jax.numpy as jnp
from jax import lax
from jax.experimental import pallas as pl
from jax.experimental.pallas import tpu as pltpu
from jax.experimental.pallas import tpu_sc as plsc

N = 10000
E = 320000
D = 128

NC = 2
NS = 16
NW = NC * NS
CH = 128
NCHUNK = 80
EPAD = NW * NCHUNK * CH
NBUF = 1
NACC = N + 256
RMAIN = 624
RTAIL_BASE = NS * RMAIN

_SC_MESH = plsc.VectorSubcoreMesh(
    core_axis_name="c", subcore_axis_name="s", num_cores=NC, num_subcores=NS)


def _tilewise_copy(src, dsti, s):
  pltpu.sync_copy(src.at[pl.ds(s * RMAIN, RMAIN)],
                  dsti.at[pl.ds(s * RMAIN, RMAIN)])

  @pl.when(s < 2)
  def _():
    pltpu.sync_copy(src.at[pl.ds(RTAIL_BASE + s * 8, 8)],
                    dsti.at[pl.ds(RTAIL_BASE + s * 8, 8)])


def _deg_body(dst3_hbm, ones_hbm, out0_hbm, out1_hbm,
              didx2, ones_v, buf_v, acc_sh):
  c = lax.axis_index("c")
  s = lax.axis_index("s")
  wid = c * NS + s

  def zfill(i, carry):
    buf_v[pl.ds(i * 16, 16)] = jnp.zeros((16,), jnp.float32)
    return carry

  lax.fori_loop(0, RMAIN // 16, zfill, 0)
  pltpu.sync_copy(buf_v, acc_sh.at[pl.ds(s * RMAIN, RMAIN)])

  @pl.when(s < 2)
  def _():
    pltpu.sync_copy(buf_v.at[pl.ds(0, 8)],
                    acc_sh.at[pl.ds(RTAIL_BASE + s * 8, 8)])

  pltpu.sync_copy(dst3_hbm.at[wid], didx2)
  pltpu.sync_copy(ones_hbm, ones_v)
  plsc.subcore_barrier()

  def chunk(i, carry):
    pltpu.sync_copy(ones_v, acc_sh.at[didx2.at[i]], add=True)
    return carry

  lax.fori_loop(0, NCHUNK, chunk, 0)
  plsc.subcore_barrier()

  @pl.when(c == 0)
  def _():
    pltpu.sync_copy(acc_sh.at[pl.ds(s * RMAIN, RMAIN)], buf_v)
    pltpu.sync_copy(buf_v, out0_hbm.at[pl.ds(s * RMAIN, RMAIN)])

    @pl.when(s < 2)
    def _():
      pltpu.sync_copy(acc_sh.at[pl.ds(RTAIL_BASE + s * 8, 8)],
                      buf_v.at[pl.ds(0, 8)])
      pltpu.sync_copy(buf_v.at[pl.ds(0, 8)],
                      out0_hbm.at[pl.ds(RTAIL_BASE + s * 8, 8)])

  @pl.when(c == 1)
  def _():
    pltpu.sync_copy(acc_sh.at[pl.ds(s * RMAIN, RMAIN)], buf_v)
    pltpu.sync_copy(buf_v, out1_hbm.at[pl.ds(s * RMAIN, RMAIN)])

    @pl.when(s < 2)
    def _():
      pltpu.sync_copy(acc_sh.at[pl.ds(RTAIL_BASE + s * 8, 8)],
                      buf_v.at[pl.ds(0, 8)])
      pltpu.sync_copy(buf_v.at[pl.ds(0, 8)],
                      out1_hbm.at[pl.ds(RTAIL_BASE + s * 8, 8)])


_deg_call = pl.kernel(
    _deg_body,
    out_type=(jax.ShapeDtypeStruct((N,), jnp.float32),
              jax.ShapeDtypeStruct((N,), jnp.float32)),
    mesh=_SC_MESH,
    scratch_types=[
        pltpu.VMEM((NCHUNK, CH), jnp.int32),
        pltpu.VMEM((CH,), jnp.float32),
        pltpu.VMEM((RMAIN,), jnp.float32),
        pltpu.VMEM_SHARED((NACC,), jnp.float32),
    ],
)


def _agg_body(hp_hbm, src3_hbm, dst3_hbm, zeros_hbm, out_hbm,
              sidx2, didx2, rows_bufs, sems, acc_sh):
  c = lax.axis_index("c")
  s = lax.axis_index("s")
  wid = c * NS + s
  pltpu.sync_copy(src3_hbm.at[wid], sidx2)
  pltpu.sync_copy(dst3_hbm.at[wid], didx2)
  _tilewise_copy(zeros_hbm, acc_sh, s)
  plsc.subcore_barrier()

  for b in range(NBUF):
    pltpu.async_copy(hp_hbm.at[sidx2.at[b]], rows_bufs[b], sems[b])

  def outer(g, carry):
    j0 = g * NBUF
    for b in range(NBUF):
      j = j0 + b
      rb = rows_bufs[b]
      pltpu.make_async_copy(hp_hbm.at[sidx2.at[j]], rb, sems[b]).wait()
      pltpu.sync_copy(rb, acc_sh.at[didx2.at[j]], add=True)

      @pl.when(j + NBUF < NCHUNK)
      def _():
        pltpu.async_copy(hp_hbm.at[sidx2.at[j + NBUF]], rb, sems[b])
    return carry

  lax.fori_loop(0, NCHUNK // NBUF, outer, 0)
  plsc.subcore_barrier()
  _tilewise_copy(acc_sh, out_hbm.at[c], s)


_agg_call = pl.kernel(
    _agg_body,
    out_type=jax.ShapeDtypeStruct((NC, N, D), jnp.float32),
    mesh=_SC_MESH,
    scratch_types=[
        pltpu.VMEM((NCHUNK, CH), jnp.int32),
        pltpu.VMEM((NCHUNK, CH), jnp.int32),
        tuple(pltpu.VMEM((CH, D), jnp.float32) for _ in range(NBUF)),
        tuple(pltpu.SemaphoreType.DMA for _ in range(NBUF)),
        pltpu.VMEM_SHARED((NACC, D), jnp.float32),
    ],
)


_MM = functools.partial(jnp.dot, precision=lax.Precision.HIGHEST,
                        preferred_element_type=jnp.float32)


def _dinv(deg_ref):
  return lax.rsqrt(1.0 + deg_ref[...])


def _first_body(x_ref, w_ref, deg_ref, o_ref):
  o_ref[...] = _MM(x_ref[...], w_ref[...]) * _dinv(deg_ref)


def _mid_body(a_ref, hp_ref, deg_ref, b_ref, w_ref, o_ref):
  dinv = _dinv(deg_ref)
  pre = (a_ref[0] + a_ref[1] + hp_ref[...]) * dinv + b_ref[...]
  act = jnp.maximum(pre, 0.0)
  o_ref[...] = _MM(act, w_ref[...]) * dinv


def _last_body(a_ref, hp_ref, deg_ref, b_ref, o_ref):
  pre = (a_ref[0] + a_ref[1] + hp_ref[...]) * _dinv(deg_ref) + b_ref[...]
  m = jnp.max(pre, axis=1, keepdims=True)
  e = jnp.exp(pre - m)
  o_ref[...] = e / jnp.sum(e, axis=1, keepdims=True)


_f32 = jnp.float32
_first_call = pl.pallas_call(
    _first_body, out_shape=jax.ShapeDtypeStruct((N, D), _f32))
_mid_call = pl.pallas_call(
    _mid_body, out_shape=jax.ShapeDtypeStruct((N, D), _f32))
_last_call = pl.pallas_call(
    _last_body, out_shape=jax.ShapeDtypeStruct((N, D), _f32))


def kernel(x, edge_idx, W1, b1, W2, b2, W3, b3, W4, b4):
  src = edge_idx[0]
  dst = edge_idx[1]
  zerosD = jnp.zeros((N, D), jnp.float32)
  ones1 = jnp.ones((CH,), jnp.float32)

  epw = E // NW
  ndum = NCHUNK * CH - epw
  widx = jnp.arange(NW, dtype=jnp.int32) % NS
  dum_dst = (N + widx[:, None] * 16
             + (jnp.arange(ndum, dtype=jnp.int32) % 16)[None, :])
  dum_src = (widx[:, None] * ndum
             + jnp.arange(ndum, dtype=jnp.int32)[None, :]) % N
  src3 = jnp.concatenate(
      [src.reshape(NW, epw), dum_src],
      axis=1).reshape(NW, NCHUNK, CH)
  dst3 = jnp.concatenate(
      [dst.reshape(NW, epw), dum_dst],
      axis=1).reshape(NW, NCHUNK, CH)

  deg0, deg1 = _deg_call(dst3, ones1)
  deg = (deg0 + deg1).reshape(N, 1)

  h1 = _first_call(x, W1, deg)
  a1 = _agg_call(h1, src3, dst3, zerosD)
  h2 = _mid_call(a1, h1, deg, b1.reshape(1, D), W2)
  a2 = _agg_call(h2, src3, dst3, zerosD)
  h3 = _mid_call(a2, h2, deg, b2.reshape(1, D), W3)
  a3 = _agg_call(h3, src3, dst3, zerosD)
  h4 = _mid_call(a3, h3, deg, b3.reshape(1, D), W4)
  a4 = _agg_call(h4, src3, dst3, zerosD)
  return _last_call(a4, h4, deg, b4.reshape(1, D))
